# baseline (speedup 1.0000x reference)
import functools

import jax
import jax.numpy as jnp
from jax import lax
from jax.experimental import pallas as pl
from jax.experimental.pallas import tpu as pltpu
from jax.experimental.pallas import tpu_sc as plsc

NC = 2
NS = 16
L = 16
NW = NC * NS
BATCH = 16384
F = 32
PACK = 128 // F
BPW = BATCH // NW
NG = 4
GR = BPW // NG
GCHUNKS = GR // L


def _mf_body(user_hbm, movie_hbm, uf_hbm, mf_hbm, out_hbm,
             uidx_v, midx_v, uridx_v, mridx_v,
             ur0, ur1, mr0, mr1, out_v,
             su0, su1, sm0, sm1):
    wid = lax.axis_index("s") * NC + lax.axis_index("c")
    base = wid * BPW

    pltpu.sync_copy(user_hbm.at[pl.ds(base, BPW)], uidx_v)
    pltpu.sync_copy(movie_hbm.at[pl.ds(base, BPW)], midx_v)

    for i in range(BPW // L):
        sl = pl.ds(i * L, L)
        uridx_v[sl] = lax.shift_right_logical(uidx_v[sl], 2)
        mridx_v[sl] = lax.shift_right_logical(midx_v[sl], 2)

    ubufs = (ur0, ur1)
    mbufs = (mr0, mr1)
    usems = (su0, su1)
    msems = (sm0, sm1)

    def start(g):
        b = g % 2
        cu = pltpu.async_copy(
            uf_hbm.at[uridx_v.at[pl.ds(g * GR, GR)]], ubufs[b], usems[b])
        cm = pltpu.async_copy(
            mf_hbm.at[mridx_v.at[pl.ds(g * GR, GR)]], mbufs[b], msems[b])
        return cu, cm

    iota = lax.iota(jnp.int32, L)
    pending = start(0)

    for g in range(NG):
        nxt = start(g + 1) if g + 1 < NG else None
        pending[0].wait()
        pending[1].wait()
        b = g % 2
        ur, mr = ubufs[b], mbufs[b]

        def chunk_body(c, carry, g=g, ur=ur, mr=mr):
            sl = pl.ds(g * GR + c * L, L)
            uoff = (uidx_v[sl] & (PACK - 1)) * F
            moff = (midx_v[sl] & (PACK - 1)) * F
            rows = c * L + iota
            acc = jnp.zeros((L,), jnp.float32)
            for f in range(F):
                uv = plsc.load_gather(ur, [rows, uoff + f])
                mv = plsc.load_gather(mr, [rows, moff + f])
                acc = acc + uv * mv
            out_v[sl] = acc
            return carry

        lax.fori_loop(0, GCHUNKS, chunk_body, 0)
        pending = nxt

    pltpu.sync_copy(out_v, out_hbm.at[pl.ds(base, BPW)])


@jax.jit
def kernel(user, movie, user_factors, movie_factors):
    n_users, n_factors = user_factors.shape
    n_movies, _ = movie_factors.shape
    uf4 = user_factors.reshape(n_users // PACK, PACK * n_factors)
    mf4 = movie_factors.reshape(n_movies // PACK, PACK * n_factors)
    mesh = plsc.VectorSubcoreMesh(
        core_axis_name="c", subcore_axis_name="s",
        num_cores=NC, num_subcores=NS)
    run = pl.kernel(
        _mf_body,
        out_type=jax.ShapeDtypeStruct((BATCH,), jnp.float32),
        mesh=mesh,
        scratch_types=[
            pltpu.VMEM((BPW,), jnp.int32),
            pltpu.VMEM((BPW,), jnp.int32),
            pltpu.VMEM((BPW,), jnp.int32),
            pltpu.VMEM((BPW,), jnp.int32),
            pltpu.VMEM((GR, PACK * F), jnp.float32),
            pltpu.VMEM((GR, PACK * F), jnp.float32),
            pltpu.VMEM((GR, PACK * F), jnp.float32),
            pltpu.VMEM((GR, PACK * F), jnp.float32),
            pltpu.VMEM((BPW,), jnp.float32),
            pltpu.SemaphoreType.DMA,
            pltpu.SemaphoreType.DMA,
            pltpu.SemaphoreType.DMA,
            pltpu.SemaphoreType.DMA,
        ],
        compiler_params=pltpu.CompilerParams(needs_layout_passes=False),
    )
    return run(user, movie, uf4, mf4)
